# baseline (speedup 1.0000x reference)
import functools

import jax
import jax.numpy as jnp
from jax import lax
from jax.experimental import pallas as pl
from jax.experimental.pallas import tpu as pltpu
from jax.experimental.pallas import tpu_sc as plsc

Q = 1024
D = 128
K = 100000
CHUNK = 128
NCHUNK = 784
KP = NCHUNK * CHUNK
BK = 2048
NKB = KP // BK
CPB = BK // CHUNK
TOPK = 16
LANES = 16
NWORKERS = 32
QPW = Q // NWORKERS
MAXV = NCHUNK // LANES
CB_CAP = 4096
CB_RED = CB_CAP - BK - 2 * LANES
NEG_INF = float("-inf")
I32_MAX = 2**31 - 1


def _tc_body(q_ref, qsq_ref, k_ref, ksq_ref, neg_ref, max_ref):
    cross = lax.dot_general(
        q_ref[...], k_ref[...],
        dimension_numbers=(((1,), (1,)), ((), ())),
        preferred_element_type=jnp.float32,
    )
    dist2 = (qsq_ref[...] + ksq_ref[...]) - 2.0 * cross
    neg = -dist2
    neg_ref[...] = neg
    max_ref[...] = jnp.max(neg.reshape(Q, CPB, CHUNK), axis=-1)[None]


def _tc_dist_and_maxima(queries, q_sq_b, keys_pad, k_sq_pad):
    return pl.pallas_call(
        _tc_body,
        grid=(NKB,),
        in_specs=[
            pl.BlockSpec((Q, D), lambda kb: (0, 0)),
            pl.BlockSpec((Q, 1), lambda kb: (0, 0)),
            pl.BlockSpec((BK, D), lambda kb: (kb, 0)),
            pl.BlockSpec((1, BK), lambda kb: (0, kb)),
        ],
        out_specs=[
            pl.BlockSpec((Q, BK), lambda kb: (0, kb)),
            pl.BlockSpec((1, Q, CPB), lambda kb: (kb, 0, 0)),
        ],
        out_shape=[
            jax.ShapeDtypeStruct((Q, KP), jnp.float32),
            jax.ShapeDtypeStruct((NKB, Q, CPB), jnp.float32),
        ],
        compiler_params=pltpu.CompilerParams(
            dimension_semantics=("arbitrary",),
        ),
    )(queries, q_sq_b, keys_pad, k_sq_pad)


HB = 16
NHB = QPW // HB
STAGE_CAP = 2 * LANES
CB_RED2 = CB_CAP - CHUNK - 2 * LANES


def _sc_select(neg_flat, max_flat):
    mesh = plsc.VectorSubcoreMesh(core_axis_name="c", subcore_axis_name="s")

    @functools.partial(
        pl.kernel,
        mesh=mesh,
        out_type=[
            jax.ShapeDtypeStruct((Q * TOPK,), jnp.float32),
            jax.ShapeDtypeStruct((Q * TOPK,), jnp.int32),
        ],
        scratch_types=[
            pltpu.VMEM((HB * NCHUNK,), jnp.float32),
            pltpu.VMEM((HB * NCHUNK,), jnp.float32),
            pltpu.VMEM((2 * LANES,), jnp.float32),
            pltpu.VMEM((2 * LANES,), jnp.int32),
            pltpu.VMEM((TOPK * 2 * LANES,), jnp.int32),
            pltpu.VMEM((STAGE_CAP * CHUNK,), jnp.float32),
            pltpu.VMEM((CB_CAP,), jnp.float32),
            pltpu.VMEM((CB_CAP,), jnp.int32),
            pltpu.VMEM((TOPK,), jnp.float32),
            pltpu.VMEM((TOPK,), jnp.int32),
            pltpu.VMEM((QPW * TOPK,), jnp.float32),
            pltpu.VMEM((QPW * TOPK,), jnp.int32),
            pltpu.VMEM((2 * LANES,), jnp.float32),
            pltpu.VMEM((2 * LANES,), jnp.int32),
            pltpu.VMEM((LANES,), jnp.float32),
            pltpu.VMEM((LANES,), jnp.int32),
            pltpu.SMEM((NCHUNK,), jnp.int32),
            pltpu.SemaphoreType.DMA,
        ],
    )
    def sc_kernel(neg_hbm, max_hbm, vals_hbm, idx_hbm,
                  mbuf, tbuf, rst, cst, ist, stage, cbv, cbi, ovals, oidx,
                  obv, obi, redf, redi, vtmp, itmp, sids, sem):
        cid = lax.axis_index("c")
        sid = lax.axis_index("s")
        wid = sid * 2 + cid
        lanes = lax.iota(jnp.int32, LANES)
        ninf = jnp.full((LANES,), NEG_INF, jnp.float32)
        pinf = jnp.full((LANES,), float("inf"), jnp.float32)
        imax = jnp.full((LANES,), I32_MAX, jnp.int32)

        def _red(v, op, ident, scratch):
            scratch[pl.ds(LANES, LANES)] = ident
            for sh in (8, 4, 2, 1):
                scratch[pl.ds(0, LANES)] = v
                v = op(v, scratch[pl.ds(sh, LANES)])
            return v[0]

        def red_f(v, op, ident):
            return _red(v, op, ident, redf)

        def red_i(v, op, ident):
            return _red(v, op, ident, redi)

        ione = jnp.full((LANES,), 1, jnp.int32)
        izero = jnp.zeros((LANES,), jnp.int32)

        def extract_cands(v, ids, tauv_s, co):
            cnt = red_i(jnp.where(v >= tauv_s, ione, izero), jnp.add,
                        izero)
            vtmp[...] = v
            itmp[...] = ids

            def ex1(co2):
                v2 = vtmp[...]
                ids2 = itmp[...]
                mx = red_f(v2, jnp.maximum, ninf)
                mxv = jnp.full((LANES,), mx, jnp.float32)
                mi = red_i(jnp.where(v2 == mxv, ids2, imax),
                           jnp.minimum, imax)
                miv = jnp.full((LANES,), mi, jnp.int32)
                cbv[pl.ds(co2, LANES)] = mxv
                cbi[pl.ds(co2, LANES)] = miv
                vtmp[...] = jnp.where((v2 == mxv) & (ids2 == miv),
                                      ninf, v2)
                return co2 + 1

            def noop(co2):
                return co2

            def rest3(co3):
                def guarded(t, co4):
                    return lax.cond(t < cnt, ex1, noop, co4)

                return lax.fori_loop(2, LANES, guarded, co3)

            def rest2(co2):
                co2 = ex1(co2)
                return lax.cond(cnt > 2, rest3, noop, co2)

            def first(co2):
                co2 = ex1(co2)
                return lax.cond(cnt > 1, rest2, noop, co2)

            return lax.cond(cnt > 0, first, noop, co)

        def select16(coff):
            cbv[pl.ds(coff, LANES)] = ninf
            cbi[pl.ds(coff, LANES)] = imax
            nv = (coff + LANES - 1) // LANES

            def pass_body(p, carry):
                ov, oi = carry

                def scan_body(t, c2):
                    bv, bi = c2
                    v = cbv[pl.ds(t * LANES, LANES)]
                    i = cbi[pl.ds(t * LANES, LANES)]
                    better = (v > bv) | ((v == bv) & (i < bi))
                    return (jnp.where(better, v, bv),
                            jnp.where(better, i, bi))

                bv, bi = lax.fori_loop(0, nv, scan_body, (ninf, imax))
                mx = red_f(bv, jnp.maximum, ninf)
                mxv = jnp.full((LANES,), mx, jnp.float32)
                mi = red_i(jnp.where(bv == mxv, bi, imax),
                           jnp.minimum, imax)
                miv = jnp.full((LANES,), mi, jnp.int32)

                def clear_body(t, _):
                    v = cbv[pl.ds(t * LANES, LANES)]
                    i = cbi[pl.ds(t * LANES, LANES)]
                    hit = (v == mxv) & (i == miv)
                    cbv[pl.ds(t * LANES, LANES)] = jnp.where(hit, ninf, v)
                    return 0

                lax.fori_loop(0, nv, clear_body, 0)
                pv = jnp.full((LANES,), p, jnp.int32)
                ov = jnp.where(lanes == pv, mxv, ov)
                oi = jnp.where(lanes == pv, miv, oi)
                return (ov, oi)

            ov, oi = lax.fori_loop(0, TOPK, pass_body, (ninf, imax))
            ovals[...] = ov
            oidx[...] = oi

        def chunk_dma(c, slot, q):
            return pltpu.make_async_copy(
                neg_hbm.at[q, pl.ds(c * CHUNK, CHUNK)],
                stage.at[pl.ds(slot * CHUNK, CHUNK)], sem)

        def process_slot(slot, c, tauv, coff):
            base = jnp.full((LANES,), c * CHUNK, jnp.int32)

            best = stage[pl.ds(slot * CHUNK, LANES)]
            bp = jnp.zeros((LANES,), jnp.int32)
            clane = jnp.where(best >= tauv, ione, izero)
            for p in range(1, CHUNK // LANES):
                v = stage[pl.ds(slot * CHUNK + p * LANES, LANES)]
                gt = v > best
                best = jnp.where(gt, v, best)
                bp = jnp.where(gt, jnp.full((LANES,), p, jnp.int32), bp)
                clane = clane + jnp.where(v >= tauv, ione, izero)
            mc = red_i(clane, jnp.maximum, izero)

            def fast(co2):
                return extract_cands(
                    best, base + bp * LANES + lanes, tauv, co2)

            def slow(co2):
                def per_vreg(p, co3):
                    v = stage[pl.ds(slot * CHUNK + p * LANES, LANES)]
                    return extract_cands(
                        v, base + (p * LANES + lanes), tauv, co3)

                return lax.fori_loop(0, CHUNK // LANES, per_vreg, co2)

            coff = lax.cond(mc >= 2, slow, fast, coff)

            def reduce_buf(co):
                select16(co)
                cbv[pl.ds(0, LANES)] = ovals[...]
                cbi[pl.ds(0, LANES)] = oidx[...]
                return jnp.int32(TOPK)

            return lax.cond(coff > CB_RED2, reduce_buf,
                            lambda co: co, coff)

        def per_query(qq, h):
            q = wid * QPW + h * HB + qq
            win = rst[pl.ds(qq, LANES)]
            m16q = win[0]
            tauv = jnp.full((LANES,), m16q, jnp.float32)
            cq = cst[pl.ds(qq, LANES)][0]

            def fetch_direct(_):
                for l in range(LANES):
                    c = ist[pl.ds(l * 2 * LANES + qq, LANES)][0]
                    chunk_dma(c, l, q).start()
                    sids[l] = c
                return jnp.int32(TOPK)

            nf = lax.cond(cq == TOPK, fetch_direct,
                          lambda _: jnp.int32(0), 0)

            def drain(j, _):
                chunk_dma(sids[j], j, q).wait()
                return 0

            lax.fori_loop(0, nf, drain, 0)

            def process(j, coff):
                return process_slot(j, sids[j], tauv, coff)

            coff = lax.fori_loop(0, nf, process, jnp.int32(0))

            def fb(co):
                def per_window(j, nfb):
                    m = mbuf[pl.ds(j * (HB * CPB) + qq * CPB, LANES)]
                    wmax = red_f(m, jnp.maximum, ninf)

                    def scan_window(n2):
                        for l in range(LANES):
                            def rec(n3, c=j * LANES + l):
                                sids[n3] = c
                                return n3 + 1

                            n2 = lax.cond(m[l] >= m16q, rec,
                                          lambda x: x, n2)
                        return n2

                    return lax.cond(wmax >= m16q, scan_window,
                                    lambda x: x, nfb)

                nfb = lax.fori_loop(0, MAXV, per_window, jnp.int32(0))

                def fbproc(j, co2):
                    c = sids[j]
                    pltpu.sync_copy(
                        neg_hbm.at[q, pl.ds(c * CHUNK, CHUNK)],
                        stage.at[pl.ds(0, CHUNK)])
                    return process_slot(0, c, tauv, co2)

                return lax.fori_loop(0, nfb, fbproc, co)

            coff = lax.cond(cq == TOPK, lambda co: co, fb, coff)
            select16(coff)
            ob = (h * HB + qq) * TOPK
            obv[pl.ds(ob, TOPK)] = ovals[...]
            obi[pl.ds(ob, TOPK)] = oidx[...]
            return h

        def half_batch(h, _):
            q0 = wid * QPW + h * HB

            def mseg(kb):
                return pltpu.make_async_copy(
                    max_hbm.at[pl.ds(kb * (Q * CPB) + q0 * CPB, HB * CPB)],
                    mbuf.at[pl.ds(kb * (HB * CPB), HB * CPB)], sem)

            def mstart(kb, _):
                mseg(kb).start()
                return 0

            def mwait(kb, _):
                mseg(kb).wait()
                return 0

            lax.fori_loop(0, NKB, mstart, 0)
            lax.fori_loop(0, NKB, mwait, 0)

            def tblock(j, _):
                xs = [mbuf[pl.ds(j * (HB * CPB) + r * CPB, LANES)]
                      for r in range(LANES)]
                for k in (1, 2, 4, 8):
                    mk = (lanes & k) != 0
                    ys = []
                    for r in range(LANES):
                        p = r ^ k
                        if r & k == 0:
                            redf[pl.ds(k, LANES)] = xs[p]
                            sh = redf[pl.ds(0, LANES)]
                            ys.append(jnp.where(mk, sh, xs[r]))
                        else:
                            redf[pl.ds(0, LANES)] = xs[p]
                            sh = redf[pl.ds(k, LANES)]
                            ys.append(jnp.where(mk, xs[r], sh))
                    xs = ys
                for r in range(LANES):
                    tbuf[pl.ds(j * (HB * CPB) + r * LANES, LANES)] = xs[r]
                return 0

            lax.fori_loop(0, MAXV, tblock, 0)

            def ins(c, carry):
                rs = list(carry[:TOPK])
                js = list(carry[TOPK:])
                v = tbuf[pl.ds(c * LANES, LANES)]
                idv = jnp.full((LANES,), c, jnp.int32)
                for r in range(TOPK):
                    gt = v > rs[r]
                    rs[r], v = (jnp.where(gt, v, rs[r]),
                                jnp.where(gt, rs[r], v))
                    js[r], idv = (jnp.where(gt, idv, js[r]),
                                  jnp.where(gt, js[r], idv))
                return tuple(rs) + tuple(js)

            init = tuple([ninf] * TOPK) + tuple([imax] * TOPK)
            carry = lax.fori_loop(0, NCHUNK, ins, init)
            r15 = carry[TOPK - 1]

            def cntp(c, acc):
                v = tbuf[pl.ds(c * LANES, LANES)]
                return acc + jnp.where(v >= r15, ione, izero)

            cnt = lax.fori_loop(0, NCHUNK, cntp, izero)

            rst[pl.ds(0, LANES)] = r15
            rst[pl.ds(LANES, LANES)] = r15
            cst[pl.ds(0, LANES)] = cnt
            cst[pl.ds(LANES, LANES)] = cnt
            for r in range(TOPK):
                ist[pl.ds(r * 2 * LANES, LANES)] = carry[TOPK + r]
                ist[pl.ds(r * 2 * LANES + LANES, LANES)] = carry[TOPK + r]

            lax.fori_loop(0, HB, per_query, h)
            return 0

        lax.fori_loop(0, NHB, half_batch, 0)
        out0 = wid * (QPW * TOPK)
        pltpu.sync_copy(obv, vals_hbm.at[pl.ds(out0, QPW * TOPK)])
        pltpu.sync_copy(obi, idx_hbm.at[pl.ds(out0, QPW * TOPK)])

    return sc_kernel(neg_flat, max_flat)


def kernel(queries, keys, k):
    q_sq = jnp.sum(queries * queries, axis=-1, keepdims=True)
    k_sq = jnp.sum(keys * keys, axis=-1)
    keys_pad = jnp.pad(keys, ((0, KP - K), (0, 0)))
    k_sq_pad = jnp.pad(k_sq, (0, KP - K),
                       constant_values=jnp.inf)[None, :]

    neg, maxima = _tc_dist_and_maxima(queries, q_sq, keys_pad, k_sq_pad)
    vals_flat, idx_flat = _sc_select(neg, maxima.reshape(NKB * Q * CPB))


    topk_vals = vals_flat.reshape(Q, TOPK)
    topk_idx = idx_flat.reshape(Q, TOPK)
    topk_idx = topk_idx + jnp.asarray(k - TOPK, dtype=topk_idx.dtype)
    e0 = jnp.repeat(jnp.arange(Q, dtype=jnp.int64), TOPK)
    e1 = topk_idx.reshape(-1).astype(jnp.int64)
    return topk_vals, topk_idx, e0, e1


# speedup vs baseline: 1.1928x; 1.1928x over previous
import functools

import jax
import jax.numpy as jnp
from jax import lax
from jax.experimental import pallas as pl
from jax.experimental.pallas import tpu as pltpu
from jax.experimental.pallas import tpu_sc as plsc

Q = 1024
D = 128
K = 100000
CHUNK = 128
NCHUNK = 784
KP = NCHUNK * CHUNK
BK = 2048
NKB = KP // BK
CPB = BK // CHUNK
TOPK = 16
LANES = 16
NWORKERS = 32
QPW = Q // NWORKERS
MAXV = NCHUNK // LANES
CB_CAP = 4096
CB_RED = CB_CAP - BK - 2 * LANES
NEG_INF = float("-inf")
I32_MAX = 2**31 - 1


def _tc_body(nq, q_ref, qsq_ref, k_ref, ksq_ref, neg_ref, max_ref):
    cross = lax.dot_general(
        q_ref[...], k_ref[...],
        dimension_numbers=(((1,), (1,)), ((), ())),
        preferred_element_type=jnp.float32,
    )
    dist2 = (qsq_ref[...] + ksq_ref[...]) - 2.0 * cross
    neg = -dist2
    neg_ref[...] = neg
    max_ref[...] = jnp.max(neg.reshape(nq, CPB, CHUNK), axis=-1).reshape(
        nq * CPB // CHUNK, CHUNK)


def _tc_dist_and_maxima(queries, q_sq_b, keys_pad, k_sq_pad, nq):
    return pl.pallas_call(
        functools.partial(_tc_body, nq),
        grid=(NKB,),
        in_specs=[
            pl.BlockSpec((nq, D), lambda kb: (0, 0)),
            pl.BlockSpec((nq, 1), lambda kb: (0, 0)),
            pl.BlockSpec((BK, D), lambda kb: (kb, 0)),
            pl.BlockSpec((1, BK), lambda kb: (0, kb)),
        ],
        out_specs=[
            pl.BlockSpec((nq, BK), lambda kb: (0, kb)),
            pl.BlockSpec((nq * CPB // CHUNK, CHUNK), lambda kb: (kb, 0)),
        ],
        out_shape=[
            jax.ShapeDtypeStruct((nq, KP), jnp.float32),
            jax.ShapeDtypeStruct((NKB * nq * CPB // CHUNK, CHUNK),
                                 jnp.float32),
        ],
        compiler_params=pltpu.CompilerParams(
            dimension_semantics=("arbitrary",),
        ),
    )(queries, q_sq_b, keys_pad, k_sq_pad)


HB = 16
NHB = QPW // HB
STAGE_CAP = 2 * LANES
CB_RED2 = CB_CAP - CHUNK - 2 * LANES


def _sc_select(neg_flat, max_flat, nq):
    mesh = plsc.VectorSubcoreMesh(core_axis_name="c", subcore_axis_name="s")
    qpw = nq // NWORKERS
    nhb = qpw // HB

    @functools.partial(
        pl.kernel,
        mesh=mesh,
        out_type=[
            jax.ShapeDtypeStruct((nq * TOPK,), jnp.float32),
            jax.ShapeDtypeStruct((nq * TOPK,), jnp.int32),
        ],
        scratch_types=[
            pltpu.VMEM((HB * NCHUNK,), jnp.float32),
            pltpu.VMEM((HB * NCHUNK,), jnp.float32),
            pltpu.VMEM((2 * LANES,), jnp.float32),
            pltpu.VMEM((2 * LANES,), jnp.int32),
            pltpu.VMEM((TOPK * 2 * LANES,), jnp.int32),
            pltpu.VMEM((STAGE_CAP * CHUNK,), jnp.float32),
            pltpu.VMEM((CB_CAP,), jnp.float32),
            pltpu.VMEM((CB_CAP,), jnp.int32),
            pltpu.VMEM((TOPK,), jnp.float32),
            pltpu.VMEM((TOPK,), jnp.int32),
            pltpu.VMEM((QPW * TOPK,), jnp.float32),
            pltpu.VMEM((QPW * TOPK,), jnp.int32),
            pltpu.VMEM((2 * LANES,), jnp.float32),
            pltpu.VMEM((2 * LANES,), jnp.int32),
            pltpu.VMEM((LANES,), jnp.float32),
            pltpu.VMEM((LANES,), jnp.int32),
            pltpu.SMEM((NCHUNK,), jnp.int32),
            pltpu.SemaphoreType.DMA,
        ],
    )
    def sc_kernel(neg_hbm, max_hbm, vals_hbm, idx_hbm,
                  mbuf, tbuf, rst, cst, ist, stage, cbv, cbi, ovals, oidx,
                  obv, obi, redf, redi, vtmp, itmp, sids, sem):
        cid = lax.axis_index("c")
        sid = lax.axis_index("s")
        wid = sid * 2 + cid
        lanes = lax.iota(jnp.int32, LANES)
        ninf = jnp.full((LANES,), NEG_INF, jnp.float32)
        pinf = jnp.full((LANES,), float("inf"), jnp.float32)
        imax = jnp.full((LANES,), I32_MAX, jnp.int32)

        def _red(v, op, ident, scratch):
            scratch[pl.ds(LANES, LANES)] = ident
            for sh in (8, 4, 2, 1):
                scratch[pl.ds(0, LANES)] = v
                v = op(v, scratch[pl.ds(sh, LANES)])
            return v[0]

        def red_f(v, op, ident):
            return _red(v, op, ident, redf)

        def red_i(v, op, ident):
            return _red(v, op, ident, redi)

        ione = jnp.full((LANES,), 1, jnp.int32)
        izero = jnp.zeros((LANES,), jnp.int32)

        def extract_cands(v, ids, tauv_s, co):
            cnt = red_i(jnp.where(v >= tauv_s, ione, izero), jnp.add,
                        izero)
            vtmp[...] = v
            itmp[...] = ids

            def ex1(co2):
                v2 = vtmp[...]
                ids2 = itmp[...]
                mx = red_f(v2, jnp.maximum, ninf)
                mxv = jnp.full((LANES,), mx, jnp.float32)
                mi = red_i(jnp.where(v2 == mxv, ids2, imax),
                           jnp.minimum, imax)
                miv = jnp.full((LANES,), mi, jnp.int32)
                cbv[pl.ds(co2, LANES)] = mxv
                cbi[pl.ds(co2, LANES)] = miv
                vtmp[...] = jnp.where((v2 == mxv) & (ids2 == miv),
                                      ninf, v2)
                return co2 + 1

            def noop(co2):
                return co2

            def rest3(co3):
                def guarded(t, co4):
                    return lax.cond(t < cnt, ex1, noop, co4)

                return lax.fori_loop(2, LANES, guarded, co3)

            def rest2(co2):
                co2 = ex1(co2)
                return lax.cond(cnt > 2, rest3, noop, co2)

            def first(co2):
                co2 = ex1(co2)
                return lax.cond(cnt > 1, rest2, noop, co2)

            return lax.cond(cnt > 0, first, noop, co)

        def select16(coff):
            cbv[pl.ds(coff, LANES)] = ninf
            cbi[pl.ds(coff, LANES)] = imax
            nv = (coff + LANES - 1) // LANES

            def pass_body(p, carry):
                ov, oi = carry

                def scan_body(t, c2):
                    bv, bi = c2
                    v = cbv[pl.ds(t * LANES, LANES)]
                    i = cbi[pl.ds(t * LANES, LANES)]
                    better = (v > bv) | ((v == bv) & (i < bi))
                    return (jnp.where(better, v, bv),
                            jnp.where(better, i, bi))

                bv, bi = lax.fori_loop(0, nv, scan_body, (ninf, imax))
                mx = red_f(bv, jnp.maximum, ninf)
                mxv = jnp.full((LANES,), mx, jnp.float32)
                mi = red_i(jnp.where(bv == mxv, bi, imax),
                           jnp.minimum, imax)
                miv = jnp.full((LANES,), mi, jnp.int32)

                def clear_body(t, _):
                    v = cbv[pl.ds(t * LANES, LANES)]
                    i = cbi[pl.ds(t * LANES, LANES)]
                    hit = (v == mxv) & (i == miv)
                    cbv[pl.ds(t * LANES, LANES)] = jnp.where(hit, ninf, v)
                    return 0

                lax.fori_loop(0, nv, clear_body, 0)
                pv = jnp.full((LANES,), p, jnp.int32)
                ov = jnp.where(lanes == pv, mxv, ov)
                oi = jnp.where(lanes == pv, miv, oi)
                return (ov, oi)

            ov, oi = lax.fori_loop(0, TOPK, pass_body, (ninf, imax))
            ovals[...] = ov
            oidx[...] = oi

        def chunk_dma(c, slot, q):
            return pltpu.make_async_copy(
                neg_hbm.at[q, pl.ds(c * CHUNK, CHUNK)],
                stage.at[pl.ds(slot * CHUNK, CHUNK)], sem)

        def process_slot(slot, c, tauv, coff):
            base = jnp.full((LANES,), c * CHUNK, jnp.int32)

            best = stage[pl.ds(slot * CHUNK, LANES)]
            bp = jnp.zeros((LANES,), jnp.int32)
            clane = jnp.where(best >= tauv, ione, izero)
            for p in range(1, CHUNK // LANES):
                v = stage[pl.ds(slot * CHUNK + p * LANES, LANES)]
                gt = v > best
                best = jnp.where(gt, v, best)
                bp = jnp.where(gt, jnp.full((LANES,), p, jnp.int32), bp)
                clane = clane + jnp.where(v >= tauv, ione, izero)
            mc = red_i(clane, jnp.maximum, izero)

            def fast(co2):
                return extract_cands(
                    best, base + bp * LANES + lanes, tauv, co2)

            def slow(co2):
                def per_vreg(p, co3):
                    v = stage[pl.ds(slot * CHUNK + p * LANES, LANES)]
                    return extract_cands(
                        v, base + (p * LANES + lanes), tauv, co3)

                return lax.fori_loop(0, CHUNK // LANES, per_vreg, co2)

            coff = lax.cond(mc >= 2, slow, fast, coff)

            def reduce_buf(co):
                select16(co)
                cbv[pl.ds(0, LANES)] = ovals[...]
                cbi[pl.ds(0, LANES)] = oidx[...]
                return jnp.int32(TOPK)

            return lax.cond(coff > CB_RED2, reduce_buf,
                            lambda co: co, coff)

        def per_query(qq, h):
            q = wid * qpw + h * HB + qq
            win = rst[pl.ds(qq, LANES)]
            m16q = win[0]
            tauv = jnp.full((LANES,), m16q, jnp.float32)
            cq = cst[pl.ds(qq, LANES)][0]

            def fetch_direct(_):
                for l in range(LANES):
                    c = ist[pl.ds(l * 2 * LANES + qq, LANES)][0]
                    chunk_dma(c, l, q).start()
                    sids[l] = c
                return jnp.int32(TOPK)

            nf = lax.cond(cq == TOPK, fetch_direct,
                          lambda _: jnp.int32(0), 0)

            def drain(j, _):
                chunk_dma(sids[j], j, q).wait()
                return 0

            lax.fori_loop(0, nf, drain, 0)

            def process(j, coff):
                return process_slot(j, sids[j], tauv, coff)

            coff = lax.fori_loop(0, nf, process, jnp.int32(0))

            def fb(co):
                def per_window(j, nfb):
                    m = mbuf[pl.ds(j * (HB * CPB) + qq * CPB, LANES)]
                    wmax = red_f(m, jnp.maximum, ninf)

                    def scan_window(n2):
                        for l in range(LANES):
                            def rec(n3, c=j * LANES + l):
                                sids[n3] = c
                                return n3 + 1

                            n2 = lax.cond(m[l] >= m16q, rec,
                                          lambda x: x, n2)
                        return n2

                    return lax.cond(wmax >= m16q, scan_window,
                                    lambda x: x, nfb)

                nfb = lax.fori_loop(0, MAXV, per_window, jnp.int32(0))

                def fbproc(j, co2):
                    c = sids[j]
                    pltpu.sync_copy(
                        neg_hbm.at[q, pl.ds(c * CHUNK, CHUNK)],
                        stage.at[pl.ds(0, CHUNK)])
                    return process_slot(0, c, tauv, co2)

                return lax.fori_loop(0, nfb, fbproc, co)

            coff = lax.cond(cq == TOPK, lambda co: co, fb, coff)
            select16(coff)
            ob = (h * HB + qq) * TOPK
            obv[pl.ds(ob, TOPK)] = ovals[...]
            obi[pl.ds(ob, TOPK)] = oidx[...]
            return h

        def half_batch(h, _):
            q0 = wid * qpw + h * HB

            def mseg(kb):
                return pltpu.make_async_copy(
                    max_hbm.at[pl.ds(kb * (nq * CPB) + q0 * CPB, HB * CPB)],
                    mbuf.at[pl.ds(kb * (HB * CPB), HB * CPB)], sem)

            def mstart(kb, _):
                mseg(kb).start()
                return 0

            def mwait(kb, _):
                mseg(kb).wait()
                return 0

            lax.fori_loop(0, NKB, mstart, 0)
            lax.fori_loop(0, NKB, mwait, 0)

            def tblock(j, _):
                xs = [mbuf[pl.ds(j * (HB * CPB) + r * CPB, LANES)]
                      for r in range(LANES)]
                for k in (1, 2, 4, 8):
                    mk = (lanes & k) != 0
                    ys = []
                    for r in range(LANES):
                        p = r ^ k
                        if r & k == 0:
                            redf[pl.ds(k, LANES)] = xs[p]
                            sh = redf[pl.ds(0, LANES)]
                            ys.append(jnp.where(mk, sh, xs[r]))
                        else:
                            redf[pl.ds(0, LANES)] = xs[p]
                            sh = redf[pl.ds(k, LANES)]
                            ys.append(jnp.where(mk, xs[r], sh))
                    xs = ys
                for r in range(LANES):
                    tbuf[pl.ds(j * (HB * CPB) + r * LANES, LANES)] = xs[r]
                return 0

            lax.fori_loop(0, MAXV, tblock, 0)

            def ins(c, carry):
                rs = list(carry[:TOPK])
                js = list(carry[TOPK:])
                v = tbuf[pl.ds(c * LANES, LANES)]
                idv = jnp.full((LANES,), c, jnp.int32)
                for r in range(TOPK):
                    gt = v > rs[r]
                    rs[r], v = (jnp.where(gt, v, rs[r]),
                                jnp.where(gt, rs[r], v))
                    js[r], idv = (jnp.where(gt, idv, js[r]),
                                  jnp.where(gt, js[r], idv))
                return tuple(rs) + tuple(js)

            init = tuple([ninf] * TOPK) + tuple([imax] * TOPK)
            carry = lax.fori_loop(0, NCHUNK, ins, init)
            r15 = carry[TOPK - 1]

            def cntp(c, acc):
                v = tbuf[pl.ds(c * LANES, LANES)]
                return acc + jnp.where(v >= r15, ione, izero)

            cnt = lax.fori_loop(0, NCHUNK, cntp, izero)

            rst[pl.ds(0, LANES)] = r15
            rst[pl.ds(LANES, LANES)] = r15
            cst[pl.ds(0, LANES)] = cnt
            cst[pl.ds(LANES, LANES)] = cnt
            for r in range(TOPK):
                ist[pl.ds(r * 2 * LANES, LANES)] = carry[TOPK + r]
                ist[pl.ds(r * 2 * LANES + LANES, LANES)] = carry[TOPK + r]

            lax.fori_loop(0, HB, per_query, h)
            return 0

        lax.fori_loop(0, nhb, half_batch, 0)
        out0 = wid * (qpw * TOPK)
        pltpu.sync_copy(obv.at[pl.ds(0, qpw * TOPK)],
                        vals_hbm.at[pl.ds(out0, qpw * TOPK)])
        pltpu.sync_copy(obi.at[pl.ds(0, qpw * TOPK)],
                        idx_hbm.at[pl.ds(out0, qpw * TOPK)])

    return sc_kernel(neg_flat, max_flat)


def kernel(queries, keys, k):
    q_sq = jnp.sum(queries * queries, axis=-1, keepdims=True)
    k_sq = jnp.sum(keys * keys, axis=-1)
    keys_pad = jnp.pad(keys, ((0, KP - K), (0, 0)))
    k_sq_pad = jnp.pad(k_sq, (0, KP - K),
                       constant_values=jnp.inf)[None, :]

    nq = Q // 2
    parts = []
    for hh in range(2):
        qs = queries[hh * nq:(hh + 1) * nq]
        qsq_h = q_sq[hh * nq:(hh + 1) * nq]
        neg, maxima = _tc_dist_and_maxima(qs, qsq_h, keys_pad, k_sq_pad,
                                          nq)
        parts.append(_sc_select(neg, maxima.reshape(NKB * nq * CPB), nq))

    vals_flat = jnp.concatenate([p[0] for p in parts])
    idx_flat = jnp.concatenate([p[1] for p in parts])
    topk_vals = vals_flat.reshape(Q, TOPK)
    topk_idx = idx_flat.reshape(Q, TOPK)
    topk_idx = topk_idx + jnp.asarray(k - TOPK, dtype=topk_idx.dtype)
    e0 = jnp.repeat(jnp.arange(Q, dtype=jnp.int64), TOPK)
    e1 = topk_idx.reshape(-1).astype(jnp.int64)
    return topk_vals, topk_idx, e0, e1
